# baseline (speedup 1.0000x reference)
import jax
import jax.numpy as jnp
import numpy as np
from jax.experimental import pallas as pl
from jax.experimental.pallas import tpu as pltpu

_B, _S, _D_IN, _D_OUT = 2, 2048, 2048, 2048
_E, _R = 8, 64
_RMOE = _E * _R
_SCALING = 16.0 / 64.0

_BM = 512


def _body(x_ref, wb_ref, wg_ref, wa_ref, wbl_ref, o_ref):
    xb = x_ref[...]

    logits = jax.lax.dot_general(
        xb, wg_ref[...], (((1,), (1,)), ((), ())),
        preferred_element_type=jnp.float32)
    lj = logits[:, None, :]
    le = logits[:, :, None]
    j_idx = jax.lax.broadcasted_iota(jnp.int32, (_BM, _E, _E), 2)
    e_idx = jax.lax.broadcasted_iota(jnp.int32, (_BM, _E, _E), 1)
    beats = (lj > le) | ((lj == le) & (j_idx < e_idx))
    rank = jnp.sum(beats.astype(jnp.int32), axis=2)
    intop = rank < 2
    m1 = jnp.max(logits, axis=1, keepdims=True)
    wun = jnp.where(intop, jnp.exp(logits - m1), 0.0)
    wts = wun / jnp.sum(wun, axis=1, keepdims=True)

    expand = (jax.lax.broadcasted_iota(jnp.int32, (_E, _RMOE), 1) // _R ==
              jax.lax.broadcasted_iota(jnp.int32, (_E, _RMOE), 0)
              ).astype(jnp.float32)
    wfull = jax.lax.dot_general(
        wts, expand, (((1,), (0,)), ((), ())),
        preferred_element_type=jnp.float32)

    h = jax.lax.dot_general(
        xb, wa_ref[...], (((1,), (1,)), ((), ())),
        preferred_element_type=jnp.float32)
    hw = (h * wfull).astype(jnp.bfloat16)
    lora = jax.lax.dot_general(
        hw, wbl_ref[...], (((1,), (1,)), ((), ())),
        preferred_element_type=jnp.float32)

    base = jax.lax.dot_general(
        xb, wb_ref[...], (((1,), (1,)), ((), ())),
        preferred_element_type=jnp.float32)
    o_ref[...] = base + _SCALING * lora


def kernel(x, W_base, W_gate, W_A, W_B):
    xf = x.reshape(_B * _S, _D_IN).astype(jnp.bfloat16)
    wb = W_base.astype(jnp.bfloat16)
    wg = W_gate.astype(jnp.bfloat16)
    wa = W_A.astype(jnp.bfloat16)
    wbl = W_B.astype(jnp.bfloat16)

    n_blocks = (_B * _S) // _BM
    out = pl.pallas_call(
        _body,
        grid=(n_blocks,),
        in_specs=[
            pl.BlockSpec((_BM, _D_IN), lambda i: (i, 0)),
            pl.BlockSpec((_D_OUT, _D_IN), lambda i: (0, 0)),
            pl.BlockSpec((_E, _D_IN), lambda i: (0, 0)),
            pl.BlockSpec((_RMOE, _D_IN), lambda i: (0, 0)),
            pl.BlockSpec((_D_OUT, _RMOE), lambda i: (0, 0)),
        ],
        out_specs=pl.BlockSpec((_BM, _D_OUT), lambda i: (i, 0)),
        out_shape=jax.ShapeDtypeStruct((_B * _S, _D_OUT), jnp.float32),
        compiler_params=pltpu.CompilerParams(
            dimension_semantics=("arbitrary",),
        ),
    )(xf, wb, wg, wa, wbl)
    return out.reshape(_B, _S, _D_OUT)


# speedup vs baseline: 2.1901x; 2.1901x over previous
import jax
import jax.numpy as jnp
import numpy as np
from jax.experimental import pallas as pl
from jax.experimental.pallas import tpu as pltpu

_B, _S, _D_IN, _D_OUT = 2, 2048, 2048, 2048
_E, _R = 8, 64
_RMOE = _E * _R
_SCALING = 16.0 / 64.0

_BM = 512


def _body(x_ref, wb_ref, wg_ref, wa_ref, wbl_ref, o_ref):
    xb = x_ref[...].astype(jnp.bfloat16)

    lT = jax.lax.dot_general(
        wg_ref[...], xb, (((1,), (1,)), ((), ())),
        preferred_element_type=jnp.float32)
    lj = lT[:, None, :]
    le = lT[None, :, :]
    j_idx = jax.lax.broadcasted_iota(jnp.int32, (_E, _E, _BM), 0)
    e_idx = jax.lax.broadcasted_iota(jnp.int32, (_E, _E, _BM), 1)
    beats = (lj > le) | ((lj == le) & (j_idx < e_idx))
    rank = jnp.sum(beats.astype(jnp.int32), axis=0)
    m1 = jnp.max(lT, axis=0, keepdims=True)
    wun = jnp.where(rank < 2, jnp.exp(lT - m1), 0.0)
    wtsT = wun / jnp.sum(wun, axis=0, keepdims=True)

    expand = (jax.lax.broadcasted_iota(jnp.int32, (_E, _RMOE), 1) // _R ==
              jax.lax.broadcasted_iota(jnp.int32, (_E, _RMOE), 0)
              ).astype(jnp.float32)
    wfull = jax.lax.dot_general(
        wtsT, expand, (((0,), (0,)), ((), ())),
        preferred_element_type=jnp.float32)

    h = jax.lax.dot_general(
        xb, wa_ref[...], (((1,), (1,)), ((), ())),
        preferred_element_type=jnp.float32)
    hw = (h * wfull).astype(jnp.bfloat16)
    lora = jax.lax.dot_general(
        hw, wbl_ref[...], (((1,), (1,)), ((), ())),
        preferred_element_type=jnp.float32)

    base = jax.lax.dot_general(
        xb, wb_ref[...], (((1,), (1,)), ((), ())),
        preferred_element_type=jnp.float32)
    o_ref[...] = base + _SCALING * lora


def kernel(x, W_base, W_gate, W_A, W_B):
    xf = x.reshape(_B * _S, _D_IN)
    wb = W_base.astype(jnp.bfloat16)
    wg = W_gate.astype(jnp.bfloat16)
    wa = W_A.astype(jnp.bfloat16)
    wbl = W_B.astype(jnp.bfloat16)

    n_blocks = (_B * _S) // _BM
    out = pl.pallas_call(
        _body,
        grid=(n_blocks,),
        in_specs=[
            pl.BlockSpec((_BM, _D_IN), lambda i: (i, 0)),
            pl.BlockSpec((_D_OUT, _D_IN), lambda i: (0, 0)),
            pl.BlockSpec((_E, _D_IN), lambda i: (0, 0)),
            pl.BlockSpec((_RMOE, _D_IN), lambda i: (0, 0)),
            pl.BlockSpec((_D_OUT, _RMOE), lambda i: (0, 0)),
        ],
        out_specs=pl.BlockSpec((_BM, _D_OUT), lambda i: (i, 0)),
        out_shape=jax.ShapeDtypeStruct((_B * _S, _D_OUT), jnp.float32),
        compiler_params=pltpu.CompilerParams(
            dimension_semantics=("arbitrary",),
        ),
    )(xf, wb, wg, wa, wbl)
    return out.reshape(_B, _S, _D_OUT)
